# baseline (speedup 1.0000x reference)
import functools

import jax
import jax.numpy as jnp
from jax import lax
from jax.experimental import pallas as pl
from jax.experimental.pallas import tpu as pltpu
from jax.experimental.pallas import tpu_sc as plsc

F32 = jnp.float32


def _x_body(n_ref, w_ref, x_ref):
    xm = jnp.dot(n_ref[...], w_ref[...], preferred_element_type=F32)
    x_ref[...] = jnp.concatenate([xm, jnp.zeros_like(xm)], axis=1)


def _node_linear(node_input, w_in):
    N, C = node_input.shape
    return pl.pallas_call(
        _x_body,
        out_shape=jax.ShapeDtypeStruct((N, 2 * C), F32),
    )(node_input, w_in)


def _m_body(attr_ref, w0e_ref, w1_ref, w2a_ref, w2b_ref, s0_ref, s1_ref,
            base0_ref, m_ref):
    a = attr_ref[...]
    hp = jax.lax.Precision.HIGHEST
    h = jnp.dot(a, w0e_ref[...], preferred_element_type=F32,
                precision=hp)
    h = h * jax.nn.sigmoid(h)
    h = jnp.dot(h, w1_ref[...], preferred_element_type=F32)
    h = h * jax.nn.sigmoid(h)
    f0 = jnp.dot(h, w2a_ref[...], preferred_element_type=F32)
    f1 = jnp.dot(h, w2b_ref[...], preferred_element_type=F32)
    fac0 = jnp.dot(a, s0_ref[...], preferred_element_type=F32,
                   precision=hp) + base0_ref[...]
    fac1 = jnp.dot(a, s1_ref[...], preferred_element_type=F32, precision=hp)
    m_ref[...] = jnp.stack([f0 * fac0, f1 * fac1], axis=0)


def _edge_multipliers(edge_attr, w0e, w1, w2a, w2b, s0, s1, base0):
    E = edge_attr.shape[0]
    B = 2000
    grid = E // B
    return pl.pallas_call(
        _m_body,
        grid=(grid,),
        in_specs=[
            pl.BlockSpec((B, 4), lambda i: (i, 0)),
            pl.BlockSpec((4, 64), lambda i: (0, 0)),
            pl.BlockSpec((64, 64), lambda i: (0, 0)),
            pl.BlockSpec((64, 128), lambda i: (0, 0)),
            pl.BlockSpec((64, 128), lambda i: (0, 0)),
            pl.BlockSpec((4, 128), lambda i: (0, 0)),
            pl.BlockSpec((4, 128), lambda i: (0, 0)),
            pl.BlockSpec((1, 128), lambda i: (0, 0)),
        ],
        out_specs=pl.BlockSpec((2, B, 128), lambda i: (0, i, 0)),
        out_shape=jax.ShapeDtypeStruct((2, E, 128), F32),
    )(edge_attr, w0e, w1, w2a, w2b, s0, s1, base0)


def _sc_body(E, N, x_hbm, m_hbm, src_hbm, dst_hbm, agg_hbm,
             acc, sidx, didx, xs, mv, zb,
             sem_g, sem_m, sem_i, sem_z):
    CH = 80
    per_tile = E // 16
    iters = per_tile // CH
    SLAB = 624
    ZB = 48
    tail0 = 16 * SLAB
    tail_n = N - tail0

    c = lax.axis_index("c")
    s = lax.axis_index("s")
    tile_base = s * per_tile

    def _fire_idx(t, b):
        base = tile_base + t * CH
        pltpu.async_copy(src_hbm.at[pl.ds(base, CH)], sidx.at[b], sem_i.at[b])
        pltpu.async_copy(dst_hbm.at[pl.ds(base, CH)], didx.at[b], sem_i.at[b])

    def _wait_idx(b):
        pltpu.make_async_copy(src_hbm.at[pl.ds(0, CH)], sidx.at[b],
                              sem_i.at[b]).wait()
        pltpu.make_async_copy(dst_hbm.at[pl.ds(0, CH)], didx.at[b],
                              sem_i.at[b]).wait()

    def _fire_data(t, b):
        base = tile_base + t * CH
        pltpu.async_copy(x_hbm.at[sidx.at[b]], xs.at[b], sem_g.at[b])
        pltpu.async_copy(m_hbm.at[c, pl.ds(base, CH)], mv.at[b], sem_m.at[b])

    def _wait_data(b):
        pltpu.make_async_copy(x_hbm.at[sidx.at[b]], xs.at[b], sem_g.at[b]).wait()
        pltpu.make_async_copy(m_hbm.at[c, pl.ds(0, CH)], mv.at[b],
                              sem_m.at[b]).wait()

    pltpu.sync_copy(src_hbm.at[pl.ds(tile_base, CH)], sidx.at[0])
    pltpu.sync_copy(dst_hbm.at[pl.ds(tile_base, CH)], didx.at[0])
    _fire_data(0, 0)
    _fire_idx(1, 1)

    zero16 = jnp.zeros((16,), F32)

    def _zrow(r, carry):
        for j in range(8):
            zb[r, pl.ds(j * 16, 16)] = zero16
        return carry

    lax.fori_loop(0, ZB, _zrow, 0)
    for k in range(SLAB // ZB):
        pltpu.async_copy(zb, acc.at[pl.ds(s * SLAB + k * ZB, ZB)], sem_z)
    for k in range(SLAB // ZB):
        pltpu.make_async_copy(zb, acc.at[pl.ds(0, ZB)], sem_z).wait()

    @pl.when(s == 0)
    def _zero_tail():
        pltpu.sync_copy(zb.at[pl.ds(0, tail_n)], acc.at[pl.ds(tail0, tail_n)])

    plsc.subcore_barrier()

    def _compute(b):
        def _edge(i, ecarry):
            xv = [xs[b, i, pl.ds(h * 16, 16)] for h in range(4)]
            for j in range(8):
                mv[b, i, pl.ds(j * 16, 16)] = (
                    xv[j % 4] * mv[b, i, pl.ds(j * 16, 16)])
            return ecarry

        lax.fori_loop(0, CH, _edge, 0)

    def _stage(t, b):
        _wait_idx(1 - b)
        _fire_data(t + 1, 1 - b)
        _wait_data(b)
        _compute(b)
        pltpu.sync_copy(mv.at[b], acc.at[didx.at[b]], add=True)

        @pl.when(t + 2 < iters)
        def _prefetch_idx():
            _fire_idx(t + 2, b)

    def _pair(k, carry):
        _stage(2 * k, 0)
        _stage(2 * k + 1, 1)
        return carry

    lax.fori_loop(0, (iters - 1) // 2, _pair, 0)
    _wait_data(0)
    _compute(0)
    pltpu.sync_copy(mv.at[0], acc.at[didx.at[0]], add=True)
    plsc.subcore_barrier()

    pltpu.sync_copy(acc.at[pl.ds(s * SLAB, SLAB)],
                    agg_hbm.at[c, pl.ds(s * SLAB, SLAB)])

    @pl.when(s == 0)
    def _drain_tail():
        pltpu.sync_copy(acc.at[pl.ds(tail0, tail_n)],
                        agg_hbm.at[c, pl.ds(tail0, tail_n)])


def _sc_scatter(x, m, src, dst):
    N = x.shape[0]
    E = src.shape[0]
    CH = 80
    mesh = plsc.VectorSubcoreMesh(core_axis_name="c", subcore_axis_name="s")
    kfn = pl.kernel(
        functools.partial(_sc_body, E, N),
        out_type=jax.ShapeDtypeStruct((2, N, 128), F32),
        mesh=mesh,
        scratch_types=[
            pltpu.VMEM_SHARED((N, 128), F32),
            pltpu.VMEM((2, CH), jnp.int32),
            pltpu.VMEM((2, CH), jnp.int32),
            pltpu.VMEM((2, CH, 128), F32),
            pltpu.VMEM((2, CH, 128), F32),
            pltpu.VMEM((48, 128), F32),
            pltpu.SemaphoreType.DMA((2,)),
            pltpu.SemaphoreType.DMA((2,)),
            pltpu.SemaphoreType.DMA((2,)),
            pltpu.SemaphoreType.DMA,
        ],
    )
    return kfn(x, m, src, dst)


def _out_body(agg_ref, u0_ref, u1_ref, o_ref):
    o_ref[...] = (
        jnp.dot(agg_ref[0], u0_ref[...], preferred_element_type=F32)
        + jnp.dot(agg_ref[1], u1_ref[...], preferred_element_type=F32))


def _final_linear(agg, u0, u1):
    N = agg.shape[1]
    B = 2000
    grid = N // B
    return pl.pallas_call(
        _out_body,
        grid=(grid,),
        in_specs=[
            pl.BlockSpec((2, B, 128), lambda i: (0, i, 0)),
            pl.BlockSpec((128, 256), lambda i: (0, 0)),
            pl.BlockSpec((128, 256), lambda i: (0, 0)),
        ],
        out_specs=pl.BlockSpec((B, 256), lambda i: (i, 0)),
        out_shape=jax.ShapeDtypeStruct((N, 256), F32),
    )(agg, u0, u1)


def kernel(node_input, edge_src, edge_dst, edge_attr,
           W_in, W_mlp0, W_mlp1, W_mlp2, W_out_scal, W_out_vec):
    N, C = node_input.shape
    E = edge_src.shape[0]

    inv_sqrt_c = 1.0 / jnp.sqrt(jnp.float32(C))
    x = _node_linear(node_input, W_in * inv_sqrt_c)

    w2n = W_mlp2 * (0.25 / jnp.sqrt(jnp.float32(64)))
    w0e = jnp.zeros((4, 64), F32).at[0].set(W_mlp0[0])
    w1n = W_mlp1 * (1.0 / jnp.sqrt(jnp.float32(64)))
    w2a = jnp.concatenate([w2n[:, 64:128], w2n[:, 0:64]], axis=1)
    w2b = jnp.concatenate([w2n[:, 0:64], w2n[:, 0:64]], axis=1)
    s0 = jnp.zeros((4, 128), F32).at[1, 64:128].set(1.0)
    s1 = (jnp.zeros((4, 128), F32).at[2, 0:64].set(1.0)
          .at[3, 64:128].set(1.0))
    base0 = jnp.concatenate(
        [jnp.ones((1, 64), F32), jnp.zeros((1, 64), F32)], axis=1)
    m = _edge_multipliers(edge_attr, w0e, w1n, w2a, w2b, s0, s1, base0)

    src = edge_src.astype(jnp.int32)
    dst = edge_dst.astype(jnp.int32)
    agg = _sc_scatter(x, m, src, dst)

    wsn = W_out_scal * inv_sqrt_c
    wvn = W_out_vec * inv_sqrt_c
    z64 = jnp.zeros((64, 64), F32)

    def ilv(k):
        parts = [z64, z64, z64]
        parts[k] = wvn
        return jnp.stack(parts, axis=-1).reshape(64, 192)

    u0 = jnp.concatenate(
        [jnp.concatenate([wsn, jnp.zeros((64, 192), F32)], axis=1),
         jnp.concatenate([z64, ilv(0)], axis=1)], axis=0)
    u1 = jnp.concatenate(
        [jnp.concatenate([z64, ilv(1)], axis=1),
         jnp.concatenate([z64, ilv(2)], axis=1)], axis=0)
    return _final_linear(agg, u0, u1)


# speedup vs baseline: 1.4192x; 1.4192x over previous
import functools

import jax
import jax.numpy as jnp
from jax import lax
from jax.experimental import pallas as pl
from jax.experimental.pallas import tpu as pltpu
from jax.experimental.pallas import tpu_sc as plsc

F32 = jnp.float32


def _x_body(n_ref, w_ref, x_ref):
    xm = jnp.dot(n_ref[...], w_ref[...], preferred_element_type=F32)
    x_ref[...] = jnp.concatenate([xm, jnp.zeros_like(xm)], axis=1)


def _node_linear(node_input, w_in):
    N, C = node_input.shape
    return pl.pallas_call(
        _x_body,
        out_shape=jax.ShapeDtypeStruct((N, 2 * C), F32),
    )(node_input, w_in)


def _m_body(attr_ref, w0_ref, w1_ref, w2a_ref, w2b_ref, m_ref):
    B = attr_ref.shape[0]
    a = attr_ref[...]
    h = a[:, 0:1] * w0_ref[...]
    h = h * jax.nn.sigmoid(h)
    h = jnp.dot(h, w1_ref[...], preferred_element_type=F32)
    h = h * jax.nn.sigmoid(h)
    f0 = jnp.dot(h, w2a_ref[...], preferred_element_type=F32)
    f1 = jnp.dot(h, w2b_ref[...], preferred_element_type=F32)
    lo = jax.lax.broadcasted_iota(jnp.int32, (B, 128), 1) < 64
    one = jnp.ones((B, 128), F32)
    v0 = jnp.broadcast_to(a[:, 1:2], (B, 128))
    v1 = jnp.broadcast_to(a[:, 2:3], (B, 128))
    v2 = jnp.broadcast_to(a[:, 3:4], (B, 128))
    m0 = f0 * jnp.where(lo, one, v0)
    m1 = f1 * jnp.where(lo, v1, v2)
    m_ref[...] = jnp.stack([m0, m1], axis=0)


def _edge_multipliers(edge_attr, w0, w1, w2a, w2b):
    E = edge_attr.shape[0]
    B = 2000
    grid = E // B
    return pl.pallas_call(
        _m_body,
        grid=(grid,),
        in_specs=[
            pl.BlockSpec((B, 4), lambda i: (i, 0)),
            pl.BlockSpec((1, 64), lambda i: (0, 0)),
            pl.BlockSpec((64, 64), lambda i: (0, 0)),
            pl.BlockSpec((64, 128), lambda i: (0, 0)),
            pl.BlockSpec((64, 128), lambda i: (0, 0)),
        ],
        out_specs=pl.BlockSpec((2, B, 128), lambda i: (0, i, 0)),
        out_shape=jax.ShapeDtypeStruct((2, E, 128), F32),
    )(edge_attr, w0, w1, w2a, w2b)


def _sc_body(E, N, x_hbm, m_hbm, src_hbm, dst_hbm, agg_hbm,
             acc, sidx, didx, xs, mv, zb,
             sem_g, sem_m, sem_i, sem_z):
    CH = 80
    per_tile = E // 16
    iters = per_tile // CH
    SLAB = 624
    ZB = 48
    tail0 = 16 * SLAB
    tail_n = N - tail0

    c = lax.axis_index("c")
    s = lax.axis_index("s")
    tile_base = s * per_tile

    def _fire_idx(t, b):
        base = tile_base + t * CH
        pltpu.async_copy(src_hbm.at[pl.ds(base, CH)], sidx.at[b], sem_i.at[b])
        pltpu.async_copy(dst_hbm.at[pl.ds(base, CH)], didx.at[b], sem_i.at[b])

    def _wait_idx(b):
        pltpu.make_async_copy(src_hbm.at[pl.ds(0, CH)], sidx.at[b],
                              sem_i.at[b]).wait()
        pltpu.make_async_copy(dst_hbm.at[pl.ds(0, CH)], didx.at[b],
                              sem_i.at[b]).wait()

    def _fire_data(t, b):
        base = tile_base + t * CH
        pltpu.async_copy(x_hbm.at[sidx.at[b]], xs.at[b], sem_g.at[b])
        pltpu.async_copy(m_hbm.at[c, pl.ds(base, CH)], mv.at[b], sem_m.at[b])

    def _wait_data(b):
        pltpu.make_async_copy(x_hbm.at[sidx.at[b]], xs.at[b], sem_g.at[b]).wait()
        pltpu.make_async_copy(m_hbm.at[c, pl.ds(0, CH)], mv.at[b],
                              sem_m.at[b]).wait()

    pltpu.sync_copy(src_hbm.at[pl.ds(tile_base, CH)], sidx.at[0])
    pltpu.sync_copy(dst_hbm.at[pl.ds(tile_base, CH)], didx.at[0])
    _fire_data(0, 0)
    _fire_idx(1, 1)

    zero16 = jnp.zeros((16,), F32)

    def _zrow(r, carry):
        for j in range(8):
            zb[r, pl.ds(j * 16, 16)] = zero16
        return carry

    lax.fori_loop(0, ZB, _zrow, 0)
    for k in range(SLAB // ZB):
        pltpu.async_copy(zb, acc.at[pl.ds(s * SLAB + k * ZB, ZB)], sem_z)
    for k in range(SLAB // ZB):
        pltpu.make_async_copy(zb, acc.at[pl.ds(0, ZB)], sem_z).wait()

    @pl.when(s == 0)
    def _zero_tail():
        pltpu.sync_copy(zb.at[pl.ds(0, tail_n)], acc.at[pl.ds(tail0, tail_n)])

    plsc.subcore_barrier()

    def _compute(b):
        def _edge(i, ecarry):
            xv = [xs[b, i, pl.ds(h * 16, 16)] for h in range(4)]
            for j in range(8):
                mv[b, i, pl.ds(j * 16, 16)] = (
                    xv[j % 4] * mv[b, i, pl.ds(j * 16, 16)])
            return ecarry

        lax.fori_loop(0, CH, _edge, 0)

    def _stage(t, b):
        _wait_idx(1 - b)
        _fire_data(t + 1, 1 - b)
        _wait_data(b)
        _compute(b)
        pltpu.sync_copy(mv.at[b], acc.at[didx.at[b]], add=True)

        @pl.when(t + 2 < iters)
        def _prefetch_idx():
            _fire_idx(t + 2, b)

    def _pair(k, carry):
        _stage(2 * k, 0)
        _stage(2 * k + 1, 1)
        return carry

    lax.fori_loop(0, (iters - 1) // 2, _pair, 0)
    _wait_data(0)
    _compute(0)
    pltpu.sync_copy(mv.at[0], acc.at[didx.at[0]], add=True)
    plsc.subcore_barrier()

    pltpu.sync_copy(acc.at[pl.ds(s * SLAB, SLAB)],
                    agg_hbm.at[c, pl.ds(s * SLAB, SLAB)])

    @pl.when(s == 0)
    def _drain_tail():
        pltpu.sync_copy(acc.at[pl.ds(tail0, tail_n)],
                        agg_hbm.at[c, pl.ds(tail0, tail_n)])


def _sc_scatter(x, m, src, dst):
    N = x.shape[0]
    E = src.shape[0]
    CH = 80
    mesh = plsc.VectorSubcoreMesh(core_axis_name="c", subcore_axis_name="s")
    kfn = pl.kernel(
        functools.partial(_sc_body, E, N),
        out_type=jax.ShapeDtypeStruct((2, N, 128), F32),
        mesh=mesh,
        scratch_types=[
            pltpu.VMEM_SHARED((N, 128), F32),
            pltpu.VMEM((2, CH), jnp.int32),
            pltpu.VMEM((2, CH), jnp.int32),
            pltpu.VMEM((2, CH, 128), F32),
            pltpu.VMEM((2, CH, 128), F32),
            pltpu.VMEM((48, 128), F32),
            pltpu.SemaphoreType.DMA((2,)),
            pltpu.SemaphoreType.DMA((2,)),
            pltpu.SemaphoreType.DMA((2,)),
            pltpu.SemaphoreType.DMA,
        ],
    )
    return kfn(x, m, src, dst)


def _out_body(agg_ref, u0_ref, u1_ref, o_ref):
    o_ref[...] = (
        jnp.dot(agg_ref[0], u0_ref[...], preferred_element_type=F32)
        + jnp.dot(agg_ref[1], u1_ref[...], preferred_element_type=F32))


def _final_linear(agg, u0, u1):
    N = agg.shape[1]
    B = 2000
    grid = N // B
    return pl.pallas_call(
        _out_body,
        grid=(grid,),
        in_specs=[
            pl.BlockSpec((2, B, 128), lambda i: (0, i, 0)),
            pl.BlockSpec((128, 256), lambda i: (0, 0)),
            pl.BlockSpec((128, 256), lambda i: (0, 0)),
        ],
        out_specs=pl.BlockSpec((B, 256), lambda i: (i, 0)),
        out_shape=jax.ShapeDtypeStruct((N, 256), F32),
    )(agg, u0, u1)


def kernel(node_input, edge_src, edge_dst, edge_attr,
           W_in, W_mlp0, W_mlp1, W_mlp2, W_out_scal, W_out_vec):
    N, C = node_input.shape
    E = edge_src.shape[0]

    inv_sqrt_c = 1.0 / jnp.sqrt(jnp.float32(C))
    x = _node_linear(node_input, W_in * inv_sqrt_c)

    w2n = W_mlp2 * (0.25 / jnp.sqrt(jnp.float32(64)))
    w1n = W_mlp1 * (1.0 / jnp.sqrt(jnp.float32(64)))
    w2a = jnp.concatenate([w2n[:, 64:128], w2n[:, 0:64]], axis=1)
    w2b = jnp.concatenate([w2n[:, 0:64], w2n[:, 0:64]], axis=1)
    m = _edge_multipliers(edge_attr, W_mlp0, w1n, w2a, w2b)

    src = edge_src.astype(jnp.int32)
    dst = edge_dst.astype(jnp.int32)
    agg = _sc_scatter(x, m, src, dst)

    wsn = W_out_scal * inv_sqrt_c
    wvn = W_out_vec * inv_sqrt_c
    z64 = jnp.zeros((64, 64), F32)

    def ilv(k):
        parts = [z64, z64, z64]
        parts[k] = wvn
        return jnp.stack(parts, axis=-1).reshape(64, 192)

    u0 = jnp.concatenate(
        [jnp.concatenate([wsn, jnp.zeros((64, 192), F32)], axis=1),
         jnp.concatenate([z64, ilv(0)], axis=1)], axis=0)
    u1 = jnp.concatenate(
        [jnp.concatenate([z64, ilv(1)], axis=1),
         jnp.concatenate([z64, ilv(2)], axis=1)], axis=0)
    return _final_linear(agg, u0, u1)
